# initial kernel scaffold (unmeasured)
import jax
import jax.numpy as jnp
from jax import lax
from jax.experimental import pallas as pl
from jax.experimental.pallas import tpu as pltpu


def kernel(
    x,
):
    def body(*refs):
        pass

    out_shape = jax.ShapeDtypeStruct(..., jnp.float32)
    return pl.pallas_call(body, out_shape=out_shape)(...)


# baseline (device time: 844441 ns/iter reference)
import jax
import jax.numpy as jnp
from jax import lax
from jax.experimental import pallas as pl
from jax.experimental.pallas import tpu as pltpu

N_DEV = 16
N_RIGHT = 8
N_LEFT = 7


def kernel(x):
    m_per, n = x.shape
    x16 = x.astype(jnp.bfloat16)

    def body(x_ref, out_ref, copy_sem, send_r, recv_r, send_l, recv_l):
        my = lax.axis_index("i")
        left = lax.rem(my - 1 + N_DEV, N_DEV)
        right = lax.rem(my + 1, N_DEV)

        barrier_sem = pltpu.get_barrier_semaphore()
        for nbr in (left, right):
            pl.semaphore_signal(
                barrier_sem, inc=1,
                device_id=(nbr,), device_id_type=pl.DeviceIdType.MESH,
            )
        pl.semaphore_wait(barrier_sem, 2)

        local_copy = pltpu.make_async_copy(
            x_ref, out_ref.at[pl.ds(my * m_per, m_per), :], copy_sem
        )
        local_copy.start()

        for h in range(N_RIGHT):
            c_r = lax.rem(my - h + N_DEV, N_DEV)
            src_r = x_ref if h == 0 else out_ref.at[pl.ds(c_r * m_per, m_per), :]
            rdma_r = pltpu.make_async_remote_copy(
                src_ref=src_r,
                dst_ref=out_ref.at[pl.ds(c_r * m_per, m_per), :],
                send_sem=send_r.at[h],
                recv_sem=recv_r.at[h],
                device_id=(right,),
                device_id_type=pl.DeviceIdType.MESH,
            )
            rdma_r.start()

            if h < N_LEFT:
                c_l = lax.rem(my + h, N_DEV)
                src_l = x_ref if h == 0 else out_ref.at[pl.ds(c_l * m_per, m_per), :]
                rdma_l = pltpu.make_async_remote_copy(
                    src_ref=src_l,
                    dst_ref=out_ref.at[pl.ds(c_l * m_per, m_per), :],
                    send_sem=send_l.at[h],
                    recv_sem=recv_l.at[h],
                    device_id=(left,),
                    device_id_type=pl.DeviceIdType.MESH,
                )
                rdma_l.start()
                rdma_l.wait()
            rdma_r.wait()

        local_copy.wait()

    return pl.pallas_call(
        body,
        out_shape=jax.ShapeDtypeStruct((N_DEV * m_per, n), jnp.bfloat16),
        in_specs=[pl.BlockSpec(memory_space=pltpu.VMEM)],
        out_specs=pl.BlockSpec(memory_space=pl.ANY),
        scratch_shapes=[
            pltpu.SemaphoreType.DMA,
            pltpu.SemaphoreType.DMA((N_RIGHT,)),
            pltpu.SemaphoreType.DMA((N_RIGHT,)),
            pltpu.SemaphoreType.DMA((N_LEFT,)),
            pltpu.SemaphoreType.DMA((N_LEFT,)),
        ],
        compiler_params=pltpu.CompilerParams(collective_id=0),
    )(x16)


# device time: 799491 ns/iter; 1.0562x vs baseline; 1.0562x over previous
import jax
import jax.numpy as jnp
from jax import lax
from jax.experimental import pallas as pl
from jax.experimental.pallas import tpu as pltpu

N_DEV = 16
N_ROUNDS = 8


def kernel(x):
    m_per, n = x.shape
    x16 = x.astype(jnp.bfloat16)

    def body(x_ref, out_ref, copy_sem, send_r, recv_r, send_l, recv_l):
        my = lax.axis_index("i")
        left = lax.rem(my - 1 + N_DEV, N_DEV)
        right = lax.rem(my + 1, N_DEV)

        barrier_sem = pltpu.get_barrier_semaphore()
        for nbr in (left, right):
            pl.semaphore_signal(
                barrier_sem, inc=1,
                device_id=(nbr,), device_id_type=pl.DeviceIdType.MESH,
            )
        pl.semaphore_wait(barrier_sem, 2)

        local_copy = pltpu.make_async_copy(
            x_ref, out_ref.at[pl.ds(my * m_per, m_per), :], copy_sem
        )
        local_copy.start()

        half = m_per // 2
        for h in range(N_ROUNDS):
            c_r = lax.rem(my - h + N_DEV, N_DEV)
            rows_r = m_per if h < N_ROUNDS - 1 else half
            src_r = (
                x_ref if h == 0 else out_ref.at[pl.ds(c_r * m_per, rows_r), :]
            )
            rdma_r = pltpu.make_async_remote_copy(
                src_ref=src_r,
                dst_ref=out_ref.at[pl.ds(c_r * m_per, rows_r), :],
                send_sem=send_r.at[h],
                recv_sem=recv_r.at[h],
                device_id=(right,),
                device_id_type=pl.DeviceIdType.MESH,
            )
            rdma_r.start()

            c_l = lax.rem(my + h, N_DEV)
            off_l = c_l * m_per if h < N_ROUNDS - 1 else c_l * m_per + half
            rows_l = m_per if h < N_ROUNDS - 1 else half
            src_l = x_ref if h == 0 else out_ref.at[pl.ds(off_l, rows_l), :]
            rdma_l = pltpu.make_async_remote_copy(
                src_ref=src_l,
                dst_ref=out_ref.at[pl.ds(off_l, rows_l), :],
                send_sem=send_l.at[h],
                recv_sem=recv_l.at[h],
                device_id=(left,),
                device_id_type=pl.DeviceIdType.MESH,
            )
            rdma_l.start()
            rdma_l.wait()
            rdma_r.wait()

        local_copy.wait()

    return pl.pallas_call(
        body,
        out_shape=jax.ShapeDtypeStruct((N_DEV * m_per, n), jnp.bfloat16),
        in_specs=[pl.BlockSpec(memory_space=pltpu.VMEM)],
        out_specs=pl.BlockSpec(memory_space=pl.ANY),
        scratch_shapes=[
            pltpu.SemaphoreType.DMA,
            pltpu.SemaphoreType.DMA((N_ROUNDS,)),
            pltpu.SemaphoreType.DMA((N_ROUNDS,)),
            pltpu.SemaphoreType.DMA((N_ROUNDS,)),
            pltpu.SemaphoreType.DMA((N_ROUNDS,)),
        ],
        compiler_params=pltpu.CompilerParams(collective_id=0),
    )(x16)
